# baseline (speedup 1.0000x reference)
import functools
import math

import jax
import jax.numpy as jnp
from jax import lax
from jax.experimental import pallas as pl
from jax.experimental.pallas import tpu as pltpu
from jax.experimental.pallas import tpu_sc as plsc

N = 10000
E = 320000
D = 128
D_EMB = 16
H = 64
G = 8
E8 = E // G

NC = 2
NS = 16
NW = NC * NS
B = 64
BR = B // G
NBLK = E // B
NB = NBLK // NW
NPAIR = NB // 2
NTAIL = NBLK - NB * NW
RC = 80
NRC = N // RC
RPT = -(-NRC // NS)

_LN2 = math.log(2.0)


def _mlp_body(ee_ref, ea_ref, w1_ref, w2_ref, den_ref, out_ref):
    inv_den = 1.0 / den_ref[0]
    for g in range(G):
        eg = ee_ref[:, D_EMB * g:D_EMB * (g + 1)]
        h = jnp.dot(eg, w1_ref[...], preferred_element_type=jnp.float32)
        h = h * (1.0 / math.sqrt(float(D_EMB)))
        h = jnp.logaddexp(h, 0.0) - _LN2
        w = jnp.dot(h, w2_ref[...], preferred_element_type=jnp.float32)
        w = w * (1.0 / math.sqrt(float(H)))
        w = w * ea_ref[:, g:g + 1] * inv_den
        out_ref[:, D * g:D * (g + 1)] = w


def _edge_weights(ee8, ea8, W1, W2, denominator):
    BE8 = 800
    grid = E8 // BE8
    return pl.pallas_call(
        _mlp_body,
        grid=(grid,),
        in_specs=[
            pl.BlockSpec((BE8, G * D_EMB), lambda i: (i, 0)),
            pl.BlockSpec((BE8, G), lambda i: (i, 0)),
            pl.BlockSpec((D_EMB, H), lambda i: (0, 0)),
            pl.BlockSpec((H, D), lambda i: (0, 0)),
            pl.BlockSpec(memory_space=pltpu.SMEM),
        ],
        out_specs=pl.BlockSpec((BE8, G * D), lambda i: (i, 0)),
        out_shape=jax.ShapeDtypeStruct((E8, G * D), jnp.float32),
    )(ee8, ea8, W1, W2, denominator)


def _sc_body(x_hbm, w2_hbm, src_hbm, dst_hbm, out_hbm,
             accum,
             src_v0, src_v1, dst_v0, dst_v1,
             rows0, rows1, w2v0, w2v1,
             semi0, semi1, semg0, semg1, semw0, semw1, semsc0, semsc1):
    cid = lax.axis_index("c")
    sid = lax.axis_index("s")
    wid = sid * NC + cid

    src_v = (src_v0, src_v1)
    dst_v = (dst_v0, dst_v1)
    rows = (rows0, rows1)
    w2v = (w2v0, w2v1)
    semi = (semi0, semi1)
    semg = (semg0, semg1)
    semw = (semw0, semw1)
    semsc = (semsc0, semsc1)

    def issue_idx(s, off):
        pltpu.async_copy(src_hbm.at[pl.ds(off, B)], src_v[s], semi[s])
        pltpu.async_copy(dst_hbm.at[pl.ds(off, B)], dst_v[s], semi[s])

    def wait_idx(s, off):
        pltpu.make_async_copy(src_hbm.at[pl.ds(off, B)], src_v[s], semi[s]).wait()
        pltpu.make_async_copy(dst_hbm.at[pl.ds(off, B)], dst_v[s], semi[s]).wait()

    def issue_fetch(s, off):
        pltpu.async_copy(x_hbm.at[src_v[s]], rows[s], semg[s])
        pltpu.async_copy(w2_hbm.at[pl.ds(pl.multiple_of(off // G, 8), BR)], w2v[s], semw[s])

    def wait_fetch(s, off):
        pltpu.make_async_copy(x_hbm.at[src_v[s]], rows[s], semg[s]).wait()
        pltpu.make_async_copy(w2_hbm.at[pl.ds(pl.multiple_of(off // G, 8), BR)], w2v[s], semw[s]).wait()

    def issue_scatter(s):
        pltpu.async_copy(rows[s], accum.at[dst_v[s]], semsc[s], add=True)

    def wait_scatter(s):
        pltpu.make_async_copy(rows[s], accum.at[dst_v[s]], semsc[s]).wait()

    def multiply(s):
        def mul(r, _):
            for g in range(G):
                for k in range(D // 16):
                    sl = pl.ds(k * 16, 16)
                    wsl = pl.ds(D * g + k * 16, 16)
                    rows[s][G * r + g, sl] = (rows[s][G * r + g, sl]
                                              * w2v[s][r, wsl])
            return 0

        lax.fori_loop(0, BR, mul, 0)

    zeros16 = jnp.zeros((16,), jnp.float32)

    def zrow(r, _):
        for k in range(D // 16):
            rows0[r, pl.ds(k * 16, 16)] = zeros16
        return 0

    lax.fori_loop(0, B, zrow, 0)
    for j in range(RPT):
        chunk = sid + NS * j

        @pl.when(chunk < NRC)
        def _():
            pltpu.sync_copy(rows0.at[pl.ds(0, RC)],
                            accum.at[pl.ds(chunk * RC, RC)])

    plsc.subcore_barrier()

    def boff(j):
        return (wid + NW * j) * B

    issue_idx(0, boff(0))
    wait_idx(0, boff(0))
    issue_fetch(0, boff(0))
    issue_idx(1, boff(1))

    def pair(t, _):
        off0 = boff(2 * t)
        off1 = boff(2 * t + 1)
        off2 = boff(2 * t + 2)
        off3 = boff(2 * t + 3)

        wait_idx(1, off1)
        issue_fetch(1, off1)
        wait_fetch(0, off0)
        multiply(0)
        issue_scatter(0)
        wait_fetch(1, off1)
        multiply(1)
        issue_scatter(1)
        wait_scatter(0)

        @pl.when(t < NPAIR - 1)
        def _():
            issue_idx(0, off2)
            wait_idx(0, off2)
            issue_fetch(0, off2)

        wait_scatter(1)

        @pl.when(t < NPAIR - 1)
        def _():
            issue_idx(1, off3)

        return 0

    lax.fori_loop(0, NPAIR, pair, 0)

    @pl.when(wid < NTAIL)
    def _():
        off = (NBLK - NTAIL + wid) * B
        issue_idx(0, off)
        wait_idx(0, off)
        issue_fetch(0, off)
        wait_fetch(0, off)
        multiply(0)
        issue_scatter(0)
        wait_scatter(0)

    plsc.subcore_barrier()

    for j in range(RPT):
        chunk = sid + NS * j

        @pl.when(chunk < NRC)
        def _():
            sl = pl.ds(chunk * RC, RC)
            pltpu.sync_copy(accum.at[sl], out_hbm.at[cid, sl])


def _sc_scatter(x, w2, src, dst):
    mesh = plsc.VectorSubcoreMesh(core_axis_name="c", subcore_axis_name="s")
    fn = functools.partial(
        pl.kernel,
        out_type=jax.ShapeDtypeStruct((NC, N, D), jnp.float32),
        mesh=mesh,
        scratch_types=[
            pltpu.VMEM_SHARED((N, D), jnp.float32),
            pltpu.VMEM((B,), jnp.int32),
            pltpu.VMEM((B,), jnp.int32),
            pltpu.VMEM((B,), jnp.int32),
            pltpu.VMEM((B,), jnp.int32),
            pltpu.VMEM((B, D), jnp.float32),
            pltpu.VMEM((B, D), jnp.float32),
            pltpu.VMEM((BR, G * D), jnp.float32),
            pltpu.VMEM((BR, G * D), jnp.float32),
            pltpu.SemaphoreType.DMA,
            pltpu.SemaphoreType.DMA,
            pltpu.SemaphoreType.DMA,
            pltpu.SemaphoreType.DMA,
            pltpu.SemaphoreType.DMA,
            pltpu.SemaphoreType.DMA,
            pltpu.SemaphoreType.DMA,
            pltpu.SemaphoreType.DMA,
        ],
    )(_sc_body)
    return fn(x, w2, src, dst)


def _combine_body(p_ref, o_ref):
    o_ref[...] = p_ref[0] + p_ref[1]


def _combine(partials):
    return pl.pallas_call(
        _combine_body,
        out_shape=jax.ShapeDtypeStruct((N, D), jnp.float32),
    )(partials)


def kernel(x, edge_attr, edge_embedding, edge_index, W1, W2, denominator):
    src = edge_index[1]
    dst = edge_index[0]
    ee8 = edge_embedding.reshape(E8, G * D_EMB)
    ea8 = edge_attr.reshape(E8, G)
    w2 = _edge_weights(ee8, ea8, W1, W2, denominator)
    partials = _sc_scatter(x, w2, src, dst)
    return _combine(partials)


# speedup vs baseline: 1.4143x; 1.4143x over previous
import functools
import math

import jax
import jax.numpy as jnp
from jax import lax
from jax.experimental import pallas as pl
from jax.experimental.pallas import tpu as pltpu
from jax.experimental.pallas import tpu_sc as plsc

N = 10000
E = 320000
D = 128
D_EMB = 16
H = 64

NC = 2
NS = 16
NW = NC * NS
B = 80
BE = 2560
RC = 80
NRC = N // RC
RPT = -(-NRC // NS)

CHUNKS = ((0, 63 * NW * B), (63 * NW * B, E - 63 * NW * B))

_LN2 = math.log(2.0)


def _mlp_body(ee_ref, ea_ref, w1_ref, w2_ref, den_ref, out_ref):
    h = jnp.dot(ee_ref[...], w1_ref[...], preferred_element_type=jnp.float32)
    h = h * (1.0 / math.sqrt(float(D_EMB)))
    h = jnp.logaddexp(h, 0.0) - _LN2
    w = jnp.dot(h, w2_ref[...], preferred_element_type=jnp.float32)
    w = w * (1.0 / math.sqrt(float(H)))
    out_ref[...] = w * ea_ref[...] * (1.0 / den_ref[0])


def _edge_weights(edge_embedding, edge_attr, W1, W2, denominator, base, ne):
    grid = ne // BE
    goff = base // BE
    return pl.pallas_call(
        _mlp_body,
        grid=(grid,),
        in_specs=[
            pl.BlockSpec((BE, D_EMB), lambda i: (i + goff, 0)),
            pl.BlockSpec((BE, 1), lambda i: (i + goff, 0)),
            pl.BlockSpec((D_EMB, H), lambda i: (0, 0)),
            pl.BlockSpec((H, D), lambda i: (0, 0)),
            pl.BlockSpec(memory_space=pltpu.SMEM),
        ],
        out_specs=pl.BlockSpec((BE, D), lambda i: (i, 0)),
        out_shape=jax.ShapeDtypeStruct((ne, D), jnp.float32),
    )(edge_embedding, edge_attr, W1, W2, denominator)


def _make_sc_body(base, ne):
    ew = ne // NW
    nb = ew // B
    npair = nb // 2

    def _sc_body(x_hbm, w2_hbm, src_hbm, dst_hbm, out_hbm,
                 accum,
                 src_v0, src_v1, dst_v0, dst_v1,
                 rows0, rows1, w2v0, w2v1,
                 semi0, semi1, semg0, semg1, semw0, semw1, semsc0, semsc1):
        cid = lax.axis_index("c")
        sid = lax.axis_index("s")
        wid = sid * NC + cid
        wbase = wid * ew

        src_v = (src_v0, src_v1)
        dst_v = (dst_v0, dst_v1)
        rows = (rows0, rows1)
        w2v = (w2v0, w2v1)
        semi = (semi0, semi1)
        semg = (semg0, semg1)
        semw = (semw0, semw1)
        semsc = (semsc0, semsc1)

        def issue_idx(s, off):
            pltpu.async_copy(src_hbm.at[pl.ds(base + off, B)], src_v[s], semi[s])
            pltpu.async_copy(dst_hbm.at[pl.ds(base + off, B)], dst_v[s], semi[s])

        def wait_idx(s, off):
            pltpu.make_async_copy(src_hbm.at[pl.ds(base + off, B)], src_v[s],
                                  semi[s]).wait()
            pltpu.make_async_copy(dst_hbm.at[pl.ds(base + off, B)], dst_v[s],
                                  semi[s]).wait()

        def issue_fetch(s, off):
            pltpu.async_copy(x_hbm.at[src_v[s]], rows[s], semg[s])
            pltpu.async_copy(w2_hbm.at[pl.ds(off, B)], w2v[s], semw[s])

        def wait_fetch(s, off):
            pltpu.make_async_copy(x_hbm.at[src_v[s]], rows[s], semg[s]).wait()
            pltpu.make_async_copy(w2_hbm.at[pl.ds(off, B)], w2v[s],
                                  semw[s]).wait()

        def issue_scatter(s):
            pltpu.async_copy(rows[s], accum.at[dst_v[s]], semsc[s], add=True)

        def wait_scatter(s):
            pltpu.make_async_copy(rows[s], accum.at[dst_v[s]], semsc[s]).wait()

        def multiply(s):
            def mul(e, _):
                for k in range(D // 16):
                    sl = pl.ds(k * 16, 16)
                    rows[s][e, sl] = rows[s][e, sl] * w2v[s][e, sl]
                return 0

            lax.fori_loop(0, B, mul, 0)

        zeros16 = jnp.zeros((16,), jnp.float32)

        def zrow(r, _):
            for k in range(D // 16):
                rows0[r, pl.ds(k * 16, 16)] = zeros16
            return 0

        lax.fori_loop(0, B, zrow, 0)
        for j in range(RPT):
            chunk = sid + NS * j

            @pl.when(chunk < NRC)
            def _():
                pltpu.sync_copy(rows0, accum.at[pl.ds(chunk * RC, RC)])

        plsc.subcore_barrier()

        issue_idx(0, wbase)
        wait_idx(0, wbase)
        issue_fetch(0, wbase)
        issue_idx(1, wbase + B)

        def pair(t, _):
            off0 = wbase + (2 * t) * B
            off1 = off0 + B
            off2 = off0 + 2 * B
            off3 = off0 + 3 * B

            wait_idx(1, off1)
            issue_fetch(1, off1)
            wait_fetch(0, off0)
            multiply(0)
            issue_scatter(0)
            wait_fetch(1, off1)
            multiply(1)
            issue_scatter(1)
            wait_scatter(0)

            if nb % 2 == 1:
                issue_idx(0, off2)
                wait_idx(0, off2)
                issue_fetch(0, off2)
            else:
                @pl.when(t < npair - 1)
                def _():
                    issue_idx(0, off2)
                    wait_idx(0, off2)
                    issue_fetch(0, off2)

            wait_scatter(1)

            @pl.when(t < npair - 1)
            def _():
                issue_idx(1, off3)

            return 0

        lax.fori_loop(0, npair, pair, 0)

        if nb % 2 == 1:
            tail = wbase + (nb - 1) * B
            wait_fetch(0, tail)
            multiply(0)
            issue_scatter(0)
            wait_scatter(0)

        plsc.subcore_barrier()

        for j in range(RPT):
            chunk = sid + NS * j

            @pl.when(chunk < NRC)
            def _():
                sl = pl.ds(chunk * RC, RC)
                pltpu.sync_copy(accum.at[sl], out_hbm.at[cid, sl])

    return _sc_body


def _sc_scatter(x, w2, src, dst, base, ne):
    mesh = plsc.VectorSubcoreMesh(core_axis_name="c", subcore_axis_name="s")
    fn = functools.partial(
        pl.kernel,
        out_type=jax.ShapeDtypeStruct((NC, N, D), jnp.float32),
        mesh=mesh,
        scratch_types=[
            pltpu.VMEM_SHARED((N, D), jnp.float32),
            pltpu.VMEM((B,), jnp.int32),
            pltpu.VMEM((B,), jnp.int32),
            pltpu.VMEM((B,), jnp.int32),
            pltpu.VMEM((B,), jnp.int32),
            pltpu.VMEM((B, D), jnp.float32),
            pltpu.VMEM((B, D), jnp.float32),
            pltpu.VMEM((B, D), jnp.float32),
            pltpu.VMEM((B, D), jnp.float32),
            pltpu.SemaphoreType.DMA,
            pltpu.SemaphoreType.DMA,
            pltpu.SemaphoreType.DMA,
            pltpu.SemaphoreType.DMA,
            pltpu.SemaphoreType.DMA,
            pltpu.SemaphoreType.DMA,
            pltpu.SemaphoreType.DMA,
            pltpu.SemaphoreType.DMA,
        ],
    )(_make_sc_body(base, ne))
    return fn(x, w2, src, dst)


def _combine_body(pa_ref, pb_ref, o_ref):
    o_ref[...] = (pa_ref[0] + pa_ref[1]) + (pb_ref[0] + pb_ref[1])


def _combine(pa, pb):
    return pl.pallas_call(
        _combine_body,
        out_shape=jax.ShapeDtypeStruct((N, D), jnp.float32),
    )(pa, pb)


def kernel(x, edge_attr, edge_embedding, edge_index, W1, W2, denominator):
    src = edge_index[1]
    dst = edge_index[0]
    parts = []
    for base, ne in CHUNKS:
        w2 = _edge_weights(edge_embedding, edge_attr, W1, W2, denominator,
                           base, ne)
        parts.append(_sc_scatter(x, w2, src, dst, base, ne))
    return _combine(parts[0], parts[1])


# speedup vs baseline: 1.4390x; 1.0175x over previous
import functools
import math

import jax
import jax.numpy as jnp
from jax import lax
from jax.experimental import pallas as pl
from jax.experimental.pallas import tpu as pltpu
from jax.experimental.pallas import tpu_sc as plsc

N = 10000
E = 320000
D = 128
D_EMB = 16
H = 64

NC = 2
NS = 16
NW = NC * NS
B = 80
BE = 2560
RC = 80
NRC = N // RC
RPT = -(-NRC // NS)

_U = NW * B
CHUNKS = ((0, 42 * _U), (42 * _U, 42 * _U), (84 * _U, E - 84 * _U))

_LN2 = math.log(2.0)


def _mlp_body(ee_ref, ea_ref, w1_ref, w2_ref, den_ref, out_ref):
    h = jnp.dot(ee_ref[...], w1_ref[...], preferred_element_type=jnp.float32)
    h = h * (1.0 / math.sqrt(float(D_EMB)))
    h = jnp.logaddexp(h, 0.0) - _LN2
    w = jnp.dot(h, w2_ref[...], preferred_element_type=jnp.float32)
    w = w * (1.0 / math.sqrt(float(H)))
    out_ref[...] = w * ea_ref[...] * (1.0 / den_ref[0])


def _edge_weights(edge_embedding, edge_attr, W1, W2, denominator, base, ne):
    grid = ne // BE
    goff = base // BE
    return pl.pallas_call(
        _mlp_body,
        grid=(grid,),
        in_specs=[
            pl.BlockSpec((BE, D_EMB), lambda i: (i + goff, 0)),
            pl.BlockSpec((BE, 1), lambda i: (i + goff, 0)),
            pl.BlockSpec((D_EMB, H), lambda i: (0, 0)),
            pl.BlockSpec((H, D), lambda i: (0, 0)),
            pl.BlockSpec(memory_space=pltpu.SMEM),
        ],
        out_specs=pl.BlockSpec((BE, D), lambda i: (i, 0)),
        out_shape=jax.ShapeDtypeStruct((ne, D), jnp.float32),
    )(edge_embedding, edge_attr, W1, W2, denominator)


def _make_sc_body(base, ne):
    ew = ne // NW
    nb = ew // B
    npair = nb // 2

    def _sc_body(x_hbm, w2_hbm, src_hbm, dst_hbm, out_hbm,
                 accum,
                 src_v0, src_v1, dst_v0, dst_v1,
                 rows0, rows1, w2v0, w2v1,
                 semi0, semi1, semg0, semg1, semw0, semw1, semsc0, semsc1):
        cid = lax.axis_index("c")
        sid = lax.axis_index("s")
        wid = sid * NC + cid
        wbase = wid * ew

        src_v = (src_v0, src_v1)
        dst_v = (dst_v0, dst_v1)
        rows = (rows0, rows1)
        w2v = (w2v0, w2v1)
        semi = (semi0, semi1)
        semg = (semg0, semg1)
        semw = (semw0, semw1)
        semsc = (semsc0, semsc1)

        def issue_idx(s, off):
            pltpu.async_copy(src_hbm.at[pl.ds(base + off, B)], src_v[s], semi[s])
            pltpu.async_copy(dst_hbm.at[pl.ds(base + off, B)], dst_v[s], semi[s])

        def wait_idx(s, off):
            pltpu.make_async_copy(src_hbm.at[pl.ds(base + off, B)], src_v[s],
                                  semi[s]).wait()
            pltpu.make_async_copy(dst_hbm.at[pl.ds(base + off, B)], dst_v[s],
                                  semi[s]).wait()

        def issue_fetch(s, off):
            pltpu.async_copy(x_hbm.at[src_v[s]], rows[s], semg[s])
            pltpu.async_copy(w2_hbm.at[pl.ds(off, B)], w2v[s], semw[s])

        def wait_fetch(s, off):
            pltpu.make_async_copy(x_hbm.at[src_v[s]], rows[s], semg[s]).wait()
            pltpu.make_async_copy(w2_hbm.at[pl.ds(off, B)], w2v[s],
                                  semw[s]).wait()

        def issue_scatter(s):
            pltpu.async_copy(rows[s], accum.at[dst_v[s]], semsc[s], add=True)

        def wait_scatter(s):
            pltpu.make_async_copy(rows[s], accum.at[dst_v[s]], semsc[s]).wait()

        def multiply(s):
            def mul(e, _):
                for k in range(D // 16):
                    sl = pl.ds(k * 16, 16)
                    rows[s][e, sl] = rows[s][e, sl] * w2v[s][e, sl]
                return 0

            lax.fori_loop(0, B, mul, 0)

        zeros16 = jnp.zeros((16,), jnp.float32)

        def zrow(r, _):
            for k in range(D // 16):
                rows0[r, pl.ds(k * 16, 16)] = zeros16
            return 0

        lax.fori_loop(0, B, zrow, 0)
        for j in range(RPT):
            chunk = sid + NS * j

            @pl.when(chunk < NRC)
            def _():
                pltpu.sync_copy(rows0, accum.at[pl.ds(chunk * RC, RC)])

        plsc.subcore_barrier()

        issue_idx(0, wbase)
        wait_idx(0, wbase)
        issue_fetch(0, wbase)
        issue_idx(1, wbase + B)

        def pair(t, _):
            off0 = wbase + (2 * t) * B
            off1 = off0 + B
            off2 = off0 + 2 * B
            off3 = off0 + 3 * B

            wait_idx(1, off1)
            issue_fetch(1, off1)
            wait_fetch(0, off0)
            multiply(0)
            issue_scatter(0)
            wait_fetch(1, off1)
            multiply(1)
            issue_scatter(1)
            wait_scatter(0)

            if nb % 2 == 1:
                issue_idx(0, off2)
                wait_idx(0, off2)
                issue_fetch(0, off2)
            else:
                @pl.when(t < npair - 1)
                def _():
                    issue_idx(0, off2)
                    wait_idx(0, off2)
                    issue_fetch(0, off2)

            wait_scatter(1)

            @pl.when(t < npair - 1)
            def _():
                issue_idx(1, off3)

            return 0

        lax.fori_loop(0, npair, pair, 0)

        if nb % 2 == 1:
            tail = wbase + (nb - 1) * B
            wait_fetch(0, tail)
            multiply(0)
            issue_scatter(0)
            wait_scatter(0)

        plsc.subcore_barrier()

        for j in range(RPT):
            chunk = sid + NS * j

            @pl.when(chunk < NRC)
            def _():
                sl = pl.ds(chunk * RC, RC)
                pltpu.sync_copy(accum.at[sl], out_hbm.at[cid, sl])

    return _sc_body


def _sc_scatter(x, w2, src, dst, base, ne):
    mesh = plsc.VectorSubcoreMesh(core_axis_name="c", subcore_axis_name="s")
    fn = functools.partial(
        pl.kernel,
        out_type=jax.ShapeDtypeStruct((NC, N, D), jnp.float32),
        mesh=mesh,
        scratch_types=[
            pltpu.VMEM_SHARED((N, D), jnp.float32),
            pltpu.VMEM((B,), jnp.int32),
            pltpu.VMEM((B,), jnp.int32),
            pltpu.VMEM((B,), jnp.int32),
            pltpu.VMEM((B,), jnp.int32),
            pltpu.VMEM((B, D), jnp.float32),
            pltpu.VMEM((B, D), jnp.float32),
            pltpu.VMEM((B, D), jnp.float32),
            pltpu.VMEM((B, D), jnp.float32),
            pltpu.SemaphoreType.DMA,
            pltpu.SemaphoreType.DMA,
            pltpu.SemaphoreType.DMA,
            pltpu.SemaphoreType.DMA,
            pltpu.SemaphoreType.DMA,
            pltpu.SemaphoreType.DMA,
            pltpu.SemaphoreType.DMA,
            pltpu.SemaphoreType.DMA,
        ],
    )(_make_sc_body(base, ne))
    return fn(x, w2, src, dst)


def _combine_body(pa_ref, pb_ref, pc_ref, o_ref):
    o_ref[...] = ((pa_ref[0] + pa_ref[1]) + (pb_ref[0] + pb_ref[1])
                  + (pc_ref[0] + pc_ref[1]))


def _combine(pa, pb, pc):
    return pl.pallas_call(
        _combine_body,
        out_shape=jax.ShapeDtypeStruct((N, D), jnp.float32),
    )(pa, pb, pc)


def kernel(x, edge_attr, edge_embedding, edge_index, W1, W2, denominator):
    src = edge_index[1]
    dst = edge_index[0]
    parts = []
    for base, ne in CHUNKS:
        w2 = _edge_weights(edge_embedding, edge_attr, W1, W2, denominator,
                           base, ne)
        parts.append(_sc_scatter(x, w2, src, dst, base, ne))
    return _combine(parts[0], parts[1], parts[2])
